# baseline (speedup 1.0000x reference)
import functools

import jax
import jax.numpy as jnp
from jax import lax
from jax.experimental import pallas as pl
from jax.experimental.pallas import tpu as pltpu
from jax.experimental.pallas import tpu_sc as plsc

_CH = 32
_ICH = 128


def _user_gather(xu2, tu, *, batch, dim):
    info = plsc.get_sparse_core_info()
    n_workers = info.num_cores * info.num_subcores
    b_w = batch // n_workers

    mesh = plsc.VectorSubcoreMesh(core_axis_name="c", subcore_axis_name="s")

    @functools.partial(
        pl.kernel,
        mesh=mesh,
        out_type=jax.ShapeDtypeStruct((batch, dim), jnp.float32),
        scratch_types=[
            pltpu.VMEM((b_w,), jnp.int32),
            pltpu.VMEM((b_w, dim), jnp.float32),
            pltpu.SemaphoreType.DMA,
        ],
    )
    def k(xu_hbm, tu_hbm, yu_hbm, xu_v, rows_v, sem):
        wid = lax.axis_index("s") * info.num_cores + lax.axis_index("c")
        base = wid * b_w

        pltpu.async_copy(xu_hbm.at[wid], xu_v, sem).wait()

        def body(c, carry):
            off = c * _CH
            for g in range(_CH // 16):
                vec = xu_v[pl.ds(off + g * 16, 16)]
                for l in range(16):
                    pltpu.async_copy(
                        tu_hbm.at[pl.ds(vec[l], 1)],
                        rows_v.at[pl.ds(off + g * 16 + l, 1)],
                        sem,
                    )
            return carry

        lax.fori_loop(0, b_w // _CH, body, 0)
        pltpu.make_async_copy(
            yu_hbm.at[pl.ds(base, b_w)], rows_v, sem
        ).wait()
        pltpu.async_copy(rows_v, yu_hbm.at[pl.ds(base, b_w)], sem).wait()

    return k(xu2, tu)


def _item_gather(xi2, ti, *, batch, dim):
    info = plsc.get_sparse_core_info()
    n_workers = info.num_cores * info.num_subcores
    b_w = batch // n_workers
    n_ch = b_w // _ICH

    mesh = plsc.VectorSubcoreMesh(core_axis_name="c", subcore_axis_name="s")

    @functools.partial(
        pl.kernel,
        mesh=mesh,
        compiler_params=pltpu.CompilerParams(use_tc_tiling_on_sc=False),
        out_type=jax.ShapeDtypeStruct((batch, dim), jnp.float32),
        scratch_types=[
            pltpu.VMEM((n_ch, _ICH), jnp.int32),
            pltpu.VMEM((b_w, dim), jnp.float32),
            pltpu.SemaphoreType.DMA,
        ],
    )
    def k(xi_hbm, ti_hbm, yi_hbm, idx_v, rows_v, sem):
        wid = lax.axis_index("s") * info.num_cores + lax.axis_index("c")
        base = wid * b_w

        pltpu.sync_copy(xi_hbm.at[pl.ds(wid * n_ch, n_ch)], idx_v)
        copies = []
        for j in range(n_ch):
            copies.append(pltpu.async_copy(
                ti_hbm.at[idx_v.at[j]],
                rows_v.at[pl.ds(j * _ICH, _ICH)],
                sem,
            ))
        for c in copies:
            c.wait()
        pltpu.sync_copy(rows_v, yi_hbm.at[pl.ds(base, b_w)])

    return k(xi2, ti)


def kernel(x_user, x_item, table_user, table_item):
    batch = x_user.shape[0]
    dim = table_user.shape[1]
    info = plsc.get_sparse_core_info()
    n_workers = info.num_cores * info.num_subcores

    xu2 = x_user.astype(jnp.int32).reshape(n_workers, batch // n_workers)
    xi2 = x_item.astype(jnp.int32).reshape(batch // _ICH, _ICH)
    del xu2
    xu3 = x_user.astype(jnp.int32).reshape(batch // _ICH, _ICH)
    yu = _item_gather(xu3, table_user, batch=batch, dim=dim)
    yi = _item_gather(xi2, table_item, batch=batch, dim=dim)
    return (yu, yi)


# speedup vs baseline: 1.2758x; 1.2758x over previous
import functools

import jax
import jax.numpy as jnp
from jax import lax
from jax.experimental import pallas as pl
from jax.experimental.pallas import tpu as pltpu
from jax.experimental.pallas import tpu_sc as plsc

_CH = 32
_N_DIRECT = 128


def _embed_lookup(xu2, xi2, tu, ti, *, batch, dim):
    info = plsc.get_sparse_core_info()
    n_workers = info.num_cores * info.num_subcores
    b_w = batch // n_workers
    n_stream = b_w - _N_DIRECT

    mesh = plsc.VectorSubcoreMesh(core_axis_name="c", subcore_axis_name="s")

    @functools.partial(
        pl.kernel,
        mesh=mesh,
        out_type=(
            jax.ShapeDtypeStruct((batch, dim), jnp.float32),
            jax.ShapeDtypeStruct((batch, dim), jnp.float32),
        ),
        scratch_types=[
            pltpu.VMEM((b_w,), jnp.int32),
            pltpu.VMEM((b_w,), jnp.int32),
            pltpu.VMEM((n_stream, dim), jnp.float32),
            pltpu.SemaphoreType.DMA,
            pltpu.SemaphoreType.DMA,
        ],
    )
    def k(xu_hbm, xi_hbm, tu_hbm, ti_hbm, yu_hbm, yi_hbm,
          xu_v, xi_v, rows_v, sem_s, sem_d):
        wid = lax.axis_index("s") * info.num_cores + lax.axis_index("c")
        base = wid * b_w

        pltpu.async_copy(xu_hbm.at[wid], xu_v, sem_s).wait()
        pltpu.async_copy(xi_hbm.at[wid], xi_v, sem_s).wait()

        def fire(t_hbm, y_hbm, x_v):
            def body_stream(c, carry):
                off = c * _CH
                for g in range(_CH // 16):
                    roff = off + g * 16
                    vec = x_v[pl.ds(roff, 16)]
                    for l in range(16):
                        pltpu.async_copy(
                            t_hbm.at[pl.ds(vec[l], 1)],
                            rows_v.at[pl.ds(roff + l, 1)],
                            sem_s,
                        )
                return carry

            def body_direct(c, carry):
                off = c * _CH
                for g in range(_CH // 16):
                    roff = off + g * 16
                    vec = x_v[pl.ds(roff, 16)]
                    for l in range(16):
                        pltpu.async_copy(
                            t_hbm.at[pl.ds(vec[l], 1)],
                            y_hbm.at[pl.ds(base + roff + l, 1)],
                            sem_d,
                        )
                return carry

            lax.fori_loop(n_stream // _CH, b_w // _CH, body_direct, 0)
            lax.fori_loop(0, n_stream // _CH, body_stream, 0)

        def drain_and_store(y_hbm):
            pltpu.make_async_copy(
                y_hbm.at[pl.ds(base, n_stream)], rows_v, sem_s
            ).wait()
            pltpu.async_copy(
                rows_v, y_hbm.at[pl.ds(base, n_stream)], sem_s
            ).wait()
            pltpu.make_async_copy(
                y_hbm.at[pl.ds(base + n_stream, _N_DIRECT)],
                y_hbm.at[pl.ds(base + n_stream, _N_DIRECT)],
                sem_d,
            ).wait()

        fire(tu_hbm, yu_hbm, xu_v)
        drain_and_store(yu_hbm)
        fire(ti_hbm, yi_hbm, xi_v)
        drain_and_store(yi_hbm)

    return k(xu2, xi2, tu, ti)


def kernel(x_user, x_item, table_user, table_item):
    batch = x_user.shape[0]
    dim = table_user.shape[1]
    info = plsc.get_sparse_core_info()
    n_workers = info.num_cores * info.num_subcores

    xu2 = x_user.astype(jnp.int32).reshape(n_workers, batch // n_workers)
    xi2 = x_item.astype(jnp.int32).reshape(n_workers, batch // n_workers)
    return _embed_lookup(xu2, xi2, table_user, table_item,
                         batch=batch, dim=dim)


# speedup vs baseline: 1.6285x; 1.2765x over previous
import functools

import jax
import jax.numpy as jnp
from jax import lax
from jax.experimental import pallas as pl
from jax.experimental.pallas import tpu as pltpu
from jax.experimental.pallas import tpu_sc as plsc

_CH = 32


def _embed_lookup(xu2, xi2, tu, ti, *, batch, dim):
    info = plsc.get_sparse_core_info()
    n_workers = info.num_cores * info.num_subcores
    b_per_w = batch // n_workers
    n_ch = b_per_w // _CH

    mesh = plsc.VectorSubcoreMesh(core_axis_name="c", subcore_axis_name="s")

    @functools.partial(
        pl.kernel,
        mesh=mesh,
        out_type=(
            jax.ShapeDtypeStruct((batch, dim), jnp.float32),
            jax.ShapeDtypeStruct((batch, dim), jnp.float32),
        ),
        scratch_types=[
            pltpu.VMEM((b_per_w,), jnp.int32),
            pltpu.VMEM((b_per_w,), jnp.int32),
            pltpu.VMEM((b_per_w, dim), jnp.float32),
            pltpu.SemaphoreType.DMA,
        ],
    )
    def k(xu_hbm, xi_hbm, tu_hbm, ti_hbm, yu_hbm, yi_hbm,
          xu_v, xi_v, rows_v, sem):
        wid = lax.axis_index("s") * info.num_cores + lax.axis_index("c")
        base = wid * b_per_w

        pltpu.async_copy(xu_hbm.at[wid], xu_v, sem).wait()
        pltpu.async_copy(xi_hbm.at[wid], xi_v, sem).wait()

        def make_body(t_hbm, rows_v, x_v):
            def body(c, carry):
                off = c * _CH
                for g in range(_CH // 16):
                    vec = x_v[pl.ds(off + g * 16, 16)]
                    for l in range(16):
                        pltpu.async_copy(
                            t_hbm.at[pl.ds(vec[l], 1)],
                            rows_v.at[pl.ds(off + g * 16 + l, 1)],
                            sem,
                        )
                return carry
            return body

        def drain_and_store(y_hbm, rows_v):
            pltpu.make_async_copy(
                y_hbm.at[pl.ds(base, b_per_w)], rows_v, sem
            ).wait()
            pltpu.async_copy(
                rows_v, y_hbm.at[pl.ds(base, b_per_w)], sem
            ).wait()

        lax.fori_loop(0, n_ch, make_body(tu_hbm, rows_v, xu_v), 0)
        drain_and_store(yu_hbm, rows_v)
        lax.fori_loop(0, n_ch, make_body(ti_hbm, rows_v, xi_v), 0)
        drain_and_store(yi_hbm, rows_v)

    return k(xu2, xi2, tu, ti)


def kernel(x_user, x_item, table_user, table_item):
    batch = x_user.shape[0]
    dim = table_user.shape[1]
    info = plsc.get_sparse_core_info()
    n_workers = info.num_cores * info.num_subcores
    b_per_w = batch // n_workers

    xu2 = x_user.astype(jnp.int32).reshape(n_workers, b_per_w)
    xi2 = x_item.astype(jnp.int32).reshape(n_workers, b_per_w)
    return _embed_lookup(xu2, xi2, table_user, table_item,
                         batch=batch, dim=dim)


# speedup vs baseline: 1.6309x; 1.0014x over previous
import functools

import jax
import jax.numpy as jnp
from jax import lax
from jax.experimental import pallas as pl
from jax.experimental.pallas import tpu as pltpu
from jax.experimental.pallas import tpu_sc as plsc

_CH = 32


def _embed_lookup(xu2, xi2, tu, ti, *, batch, dim):
    info = plsc.get_sparse_core_info()
    n_workers = info.num_cores * info.num_subcores
    b_per_w = batch // n_workers
    n_ch = b_per_w // _CH

    mesh = plsc.VectorSubcoreMesh(core_axis_name="c", subcore_axis_name="s")

    @functools.partial(
        pl.kernel,
        mesh=mesh,
        out_type=(
            jax.ShapeDtypeStruct((batch, dim), jnp.float32),
            jax.ShapeDtypeStruct((batch, dim), jnp.float32),
        ),
        scratch_types=[
            pltpu.VMEM((b_per_w,), jnp.int32),
            pltpu.VMEM((b_per_w,), jnp.int32),
            pltpu.VMEM((b_per_w + b_per_w // 2, dim), jnp.float32),
            pltpu.SemaphoreType.DMA,
        ],
    )
    def k(xu_hbm, xi_hbm, tu_hbm, ti_hbm, yu_hbm, yi_hbm,
          xu_v, xi_v, rows_v, sem):
        wid = lax.axis_index("s") * info.num_cores + lax.axis_index("c")
        base = wid * b_per_w

        pltpu.async_copy(xu_hbm.at[wid], xu_v, sem).wait()
        pltpu.async_copy(xi_hbm.at[wid], xi_v, sem).wait()

        half = b_per_w // 2

        def make_body(t_hbm, x_v, x_off, buf_off):
            def body(c, carry):
                off = c * _CH
                for g in range(_CH // 16):
                    vec = x_v[pl.ds(x_off + off + g * 16, 16)]
                    for l in range(16):
                        pltpu.async_copy(
                            t_hbm.at[pl.ds(vec[l], 1)],
                            rows_v.at[pl.ds(buf_off + off + g * 16 + l, 1)],
                            sem,
                        )
                return carry
            return body

        def drain(n_rows):
            pltpu.make_async_copy(
                yu_hbm.at[pl.ds(base, n_rows)],
                rows_v.at[pl.ds(0, n_rows)],
                sem,
            ).wait()

        lax.fori_loop(0, n_ch, make_body(tu_hbm, xu_v, 0, 0), 0)
        lax.fori_loop(0, half // _CH, make_body(ti_hbm, xi_v, 0, b_per_w), 0)
        drain(b_per_w + half)
        pltpu.async_copy(
            rows_v.at[pl.ds(0, b_per_w)],
            yu_hbm.at[pl.ds(base, b_per_w)], sem,
        )
        lax.fori_loop(0, half // _CH, make_body(ti_hbm, xi_v, half, 0), 0)
        drain(b_per_w)
        drain(half)
        pltpu.async_copy(
            rows_v.at[pl.ds(b_per_w, half)],
            yi_hbm.at[pl.ds(base, half)], sem,
        ).wait()
        pltpu.async_copy(
            rows_v.at[pl.ds(0, half)],
            yi_hbm.at[pl.ds(base + half, half)], sem,
        ).wait()

    return k(xu2, xi2, tu, ti)


def kernel(x_user, x_item, table_user, table_item):
    batch = x_user.shape[0]
    dim = table_user.shape[1]
    info = plsc.get_sparse_core_info()
    n_workers = info.num_cores * info.num_subcores
    b_per_w = batch // n_workers

    xu2 = x_user.astype(jnp.int32).reshape(n_workers, b_per_w)
    xi2 = x_item.astype(jnp.int32).reshape(n_workers, b_per_w)
    return _embed_lookup(xu2, xi2, table_user, table_item,
                         batch=batch, dim=dim)
